# baseline (speedup 1.0000x reference)
import functools

import jax
import jax.numpy as jnp
from jax import lax
from jax.experimental import pallas as pl
from jax.experimental.pallas import tpu as pltpu
from jax.experimental.pallas import tpu_sc as plsc

_COMMITMENT_COST = 1.0
_DIVERSITY_WEIGHT = 0.1


def _tree8(a):
    t1 = a[0:1, :] + a[4:5, :]
    t2 = a[2:3, :] + a[6:7, :]
    t3 = a[1:2, :] + a[5:6, :]
    t4 = a[3:4, :] + a[7:8, :]
    return (t1 + t2) + (t3 + t4)


def _rowsum(x):
    c = x.shape[1]
    t = x.T
    acc = t[0:8, :]
    for i in range(1, c // 8):
        acc = acc + t[8 * i:8 * i + 8, :]
    return _tree8(acc)


def _colsum_block(x):
    g = x[0:128, :]
    for cc in range(1, 8):
        g = g + x[128 * cc:128 * cc + 128, :]
    acc = g[0:8, :]
    for lt in range(1, 16):
        acc = acc + g[8 * lt:8 * lt + 8, :]
    return _tree8(acc)


def _tc_body(ngrid, z_ref, cb_ref, idx_ref, ps_ref, mse_ref,
             cbn_ref, cn2n_ref, wsel_ref, msa_ref):
    pid = pl.program_id(0)
    k = cb_ref.shape[0]

    @pl.when(pid == 0)
    def _init():
        cbm = cb_ref[...]
        craw2 = _rowsum(cbm * cbm)
        cbn = cbm / jnp.maximum(jnp.sqrt(craw2.T), 1e-12)
        cbn_ref[...] = cbn
        cn2n = _rowsum(cbn * cbn)
        cn2n_ref[...] = cn2n
        iot = lax.broadcasted_iota(jnp.int32, (k, 1), 0).astype(jnp.float32)
        pad = jnp.zeros((k, 125), jnp.float32)
        wsel_ref[...] = jnp.concatenate([iot, craw2.T, cn2n.T, pad], axis=1)
        msa_ref[0] = 0.0

    z = z_ref[...]
    r = z.shape[0]
    ze2 = _rowsum(z * z).T
    zen = jnp.sqrt(ze2)
    zn = z / jnp.maximum(zen, 1e-12)
    zn2 = _rowsum(zn * zn).T

    s = lax.dot_general(zn, cbn_ref[...], (((1,), (1,)), ((), ())),
                        preferred_element_type=jnp.float32)
    d = (zn2 + cn2n_ref[...]) - 2.0 * s

    m = jnp.min(d, axis=1, keepdims=True)
    sel = jnp.where(d == m, 1.0, 0.0)
    g = lax.dot_general(sel, wsel_ref[...], (((1,), (0,)), ((), ())),
                        precision=lax.Precision.HIGHEST,
                        preferred_element_type=jnp.float32)
    idxf = jnp.minimum(g[:, 0:1], jnp.float32(k - 1))
    idx = idxf.astype(jnp.int32)
    idx_ref[0, 0, :] = idx[:, 0]

    c2_sel = g[:, 1:2]
    cn2n_sel = g[:, 2:3]
    s_sel = ((zn2 + cn2n_sel) - m) * 0.5
    mse_rows = (ze2 - 2.0 * zen * jnp.sqrt(c2_sel) * s_sel) + c2_sel
    msa_ref[0] += jnp.sum(mse_rows)

    e = jnp.exp(m - d)
    srow = _rowsum(e)
    p = e / srow.T
    t = _colsum_block(p)

    @pl.when(pid == 0)
    def _first():
        ps_ref[...] = t

    @pl.when(pid != 0)
    def _acc():
        ps_ref[...] = ps_ref[...] + t

    @pl.when(pid == ngrid - 1)
    def _fin():
        mse_ref[0, 0] = msa_ref[0]


def _tc_forward(z_e, codebook, block_rows=1024):
    n, dim = z_e.shape
    k = codebook.shape[0]
    ngrid = n // block_rows
    f32 = jnp.float32
    return pl.pallas_call(
        functools.partial(_tc_body, ngrid),
        grid=(ngrid,),
        in_specs=[
            pl.BlockSpec((block_rows, dim), lambda i: (i, 0)),
            pl.BlockSpec((k, dim), lambda i: (0, 0)),
        ],
        out_specs=[
            pl.BlockSpec((1, 1, block_rows), lambda i: (i, 0, 0)),
            pl.BlockSpec((1, k), lambda i: (0, 0)),
            pl.BlockSpec((1, 1), lambda i: (0, 0), memory_space=pltpu.SMEM),
        ],
        out_shape=[
            jax.ShapeDtypeStruct((ngrid, 1, block_rows), jnp.int32),
            jax.ShapeDtypeStruct((1, k), f32),
            jax.ShapeDtypeStruct((1, 1), f32),
        ],
        scratch_shapes=[
            pltpu.VMEM((k, dim), f32),
            pltpu.VMEM((1, k), f32),
            pltpu.VMEM((k, 128), f32),
            pltpu.SMEM((1,), f32),
        ],
    )(z_e, codebook)


def _sc_gather(codebook, idx):
    n = idx.shape[0]
    dim = codebook.shape[1]
    info = plsc.get_sparse_core_info()
    nw = info.num_cores * info.num_subcores
    b_per_w = n // nw
    chunk = 512
    nchunk = b_per_w // chunk
    mesh = plsc.VectorSubcoreMesh(core_axis_name="c", subcore_axis_name="s")

    @functools.partial(
        pl.kernel, mesh=mesh,
        out_type=jax.ShapeDtypeStruct((n, dim), jnp.float32),
        scratch_types=[
            pltpu.VMEM((chunk,), jnp.int32),
            pltpu.VMEM((chunk, dim), jnp.float32),
            pltpu.SemaphoreType.DMA,
        ],
    )
    def gather(cb_hbm, idx_hbm, out_hbm, idx_v, rows_v, sem):
        wid = lax.axis_index("s") * info.num_cores + lax.axis_index("c")
        base = wid * b_per_w
        for t in range(nchunk):
            off = base + t * chunk
            pltpu.sync_copy(idx_hbm.at[pl.ds(off, chunk)], idx_v)
            pltpu.async_copy(cb_hbm.at[idx_v], rows_v, sem).wait()
            pltpu.sync_copy(rows_v, out_hbm.at[pl.ds(off, chunk)])

    return gather(codebook, idx)


def kernel(z_e, codebook):
    n, dim = z_e.shape
    k = codebook.shape[0]
    idx3, psum, msesum = _tc_forward(z_e, codebook)
    idx = idx3.reshape(-1)
    z_q = _sc_gather(codebook, idx)

    avg = psum[0] / jnp.float32(n)
    entropy = -jnp.sum(avg * jnp.log(avg + 1e-10))
    diversity = jnp.log(jnp.float32(k)) - entropy
    mse_mean = msesum[0, 0] / jnp.float32(n * dim)
    vq = (1.0 + _COMMITMENT_COST) * mse_mean
    total = vq + _DIVERSITY_WEIGHT * diversity
    return (z_q, vq, diversity, total, idx)


# speedup vs baseline: 1.5316x; 1.5316x over previous
import functools

import jax
import jax.numpy as jnp
from jax import lax
from jax.experimental import pallas as pl
from jax.experimental.pallas import tpu as pltpu
from jax.experimental.pallas import tpu_sc as plsc

_COMMITMENT_COST = 1.0
_DIVERSITY_WEIGHT = 0.1


def _tree8(a):
    t1 = a[0:1, :] + a[4:5, :]
    t2 = a[2:3, :] + a[6:7, :]
    t3 = a[1:2, :] + a[5:6, :]
    t4 = a[3:4, :] + a[7:8, :]
    return (t1 + t2) + (t3 + t4)


def _rowsum(x):
    c = x.shape[1]
    t = x.T
    acc = t[0:8, :]
    for i in range(1, c // 8):
        acc = acc + t[8 * i:8 * i + 8, :]
    return _tree8(acc)


def _colsum_block(x):
    g = x[0:128, :]
    for cc in range(1, 8):
        g = g + x[128 * cc:128 * cc + 128, :]
    acc = g[0:8, :]
    for lt in range(1, 16):
        acc = acc + g[8 * lt:8 * lt + 8, :]
    return _tree8(acc)


def _tc_body(ngrid, z_ref, cb_ref, idx_ref, ps_ref, mse_ref,
             cbn_ref, cn2n_ref, wsel_ref, msa_ref):
    pid = pl.program_id(0)
    k = cb_ref.shape[0]

    @pl.when(pid == 0)
    def _init():
        cbm = cb_ref[...]
        craw2 = _rowsum(cbm * cbm)
        cbn = cbm / jnp.maximum(jnp.sqrt(craw2.T), 1e-12)
        cbn_ref[...] = cbn
        cn2n = _rowsum(cbn * cbn)
        cn2n_ref[...] = cn2n
        iot = lax.broadcasted_iota(jnp.int32, (k, 1), 0)
        hi = (iot // 32).astype(jnp.float32)
        lo = (iot % 32).astype(jnp.float32)
        pad = jnp.zeros((k, 4), jnp.float32)
        wsel_ref[...] = jnp.concatenate([hi, lo, craw2.T, cn2n.T, pad],
                                        axis=1)
        msa_ref[0] = 0.0

    z = z_ref[...]
    r = z.shape[0]
    ze2 = _rowsum(z * z).T
    zen = jnp.sqrt(ze2)
    zn = z / jnp.maximum(zen, 1e-12)
    zn2 = _rowsum(zn * zn).T

    s = lax.dot_general(zn, cbn_ref[...], (((1,), (1,)), ((), ())),
                        preferred_element_type=jnp.float32)
    d = (zn2 + cn2n_ref[...]) - 2.0 * s

    m = jnp.min(d, axis=1, keepdims=True)
    sel = jnp.where(d == m, 1.0, 0.0)
    g = lax.dot_general(sel, wsel_ref[...], (((1,), (0,)), ((), ())),
                        preferred_element_type=jnp.float32)
    idxf = g[:, 0:1] * 32.0 + g[:, 1:2]
    idx = jnp.minimum(idxf, jnp.float32(k - 1)).astype(jnp.int32)
    idx_ref[0, 0, :] = idx[:, 0]

    c2_sel = g[:, 2:3]
    cn2n_sel = g[:, 3:4]
    s_sel = ((zn2 + cn2n_sel) - m) * 0.5
    mse_rows = (ze2 - 2.0 * zen * jnp.sqrt(c2_sel) * s_sel) + c2_sel
    msa_ref[0] += jnp.sum(mse_rows)

    e = jnp.exp(m - d)
    srow = _rowsum(e)
    p = e / srow.T
    t = _colsum_block(p)

    @pl.when(pid == 0)
    def _first():
        ps_ref[...] = t

    @pl.when(pid != 0)
    def _acc():
        ps_ref[...] = ps_ref[...] + t

    @pl.when(pid == ngrid - 1)
    def _fin():
        mse_ref[0, 0] = msa_ref[0]


def _tc_forward(z_e, codebook, block_rows=1024):
    n, dim = z_e.shape
    k = codebook.shape[0]
    ngrid = n // block_rows
    f32 = jnp.float32
    return pl.pallas_call(
        functools.partial(_tc_body, ngrid),
        grid=(ngrid,),
        in_specs=[
            pl.BlockSpec((block_rows, dim), lambda i: (i, 0)),
            pl.BlockSpec((k, dim), lambda i: (0, 0)),
        ],
        out_specs=[
            pl.BlockSpec((1, 1, block_rows), lambda i: (i, 0, 0)),
            pl.BlockSpec((1, k), lambda i: (0, 0)),
            pl.BlockSpec((1, 1), lambda i: (0, 0), memory_space=pltpu.SMEM),
        ],
        out_shape=[
            jax.ShapeDtypeStruct((ngrid, 1, block_rows), jnp.int32),
            jax.ShapeDtypeStruct((1, k), f32),
            jax.ShapeDtypeStruct((1, 1), f32),
        ],
        scratch_shapes=[
            pltpu.VMEM((k, dim), f32),
            pltpu.VMEM((1, k), f32),
            pltpu.VMEM((k, 8), f32),
            pltpu.SMEM((1,), f32),
        ],
    )(z_e, codebook)


def _sc_gather(codebook, idx):
    n = idx.shape[0]
    dim = codebook.shape[1]
    info = plsc.get_sparse_core_info()
    nw = info.num_cores * info.num_subcores
    b_per_w = n // nw
    chunk = 512
    nchunk = b_per_w // chunk
    mesh = plsc.VectorSubcoreMesh(core_axis_name="c", subcore_axis_name="s")

    @functools.partial(
        pl.kernel, mesh=mesh,
        out_type=jax.ShapeDtypeStruct((n, dim), jnp.float32),
        scratch_types=[
            pltpu.VMEM((chunk,), jnp.int32),
            pltpu.VMEM((chunk, dim), jnp.float32),
            pltpu.SemaphoreType.DMA,
        ],
    )
    def gather(cb_hbm, idx_hbm, out_hbm, idx_v, rows_v, sem):
        wid = lax.axis_index("s") * info.num_cores + lax.axis_index("c")
        base = wid * b_per_w
        for t in range(nchunk):
            off = base + t * chunk
            pltpu.sync_copy(idx_hbm.at[pl.ds(off, chunk)], idx_v)
            pltpu.async_copy(cb_hbm.at[idx_v], rows_v, sem).wait()
            pltpu.sync_copy(rows_v, out_hbm.at[pl.ds(off, chunk)])

    return gather(codebook, idx)


def kernel(z_e, codebook):
    n, dim = z_e.shape
    k = codebook.shape[0]
    idx3, psum, msesum = _tc_forward(z_e, codebook)
    idx = idx3.reshape(-1)
    z_q = _sc_gather(codebook, idx)

    avg = psum[0] / jnp.float32(n)
    entropy = -jnp.sum(avg * jnp.log(avg + 1e-10))
    diversity = jnp.log(jnp.float32(k)) - entropy
    mse_mean = msesum[0, 0] / jnp.float32(n * dim)
    vq = (1.0 + _COMMITMENT_COST) * mse_mean
    total = vq + _DIVERSITY_WEIGHT * diversity
    return (z_q, vq, diversity, total, idx)
